# Initial kernel scaffold; baseline (speedup 1.0000x reference)
import jax
import jax.numpy as jnp
from jax.experimental import pallas as pl


def kernel(x, edge_index, W_shared, b_shared, W_mu_c, b_mu_c, W_mu_nc, b_mu_nc, W_lv_c, b_lv_c, W_lv_nc, b_lv_nc, dc_fcW, dc_fcb, dc_fc2W, dc_fc2b, dn_fcW, dn_fcb, dn_fc2W, dn_fc2b):
    raise NotImplementedError("write your pallas kernel here")


# speedup vs baseline: 36.9732x; 36.9732x over previous
import functools

import jax
import jax.numpy as jnp
from jax import lax
from jax.experimental import pallas as pl
from jax.experimental.pallas import tpu as pltpu
from jax.experimental.pallas import tpu_sc as plsc

_N = 10000
_E = 320000
_NC = 2
_NS = 16
_NW = _NC * _NS
_EPT = _E // _NW
_CB = 1000
_CD = 2000
_ROWS_PER_TILE = _N // _NS

_sc_mesh = plsc.VectorSubcoreMesh(core_axis_name="c", subcore_axis_name="s")


def _deg_body(dst_hbm, zeros_hbm, out_hbm, idx_v, ones_v, acc_sh, sem):
    c = lax.axis_index("c")
    s = lax.axis_index("s")
    wid = c * _NS + s

    def fill(i, carry):
        ones_v[i, :] = jnp.full((16,), 1.0, jnp.float32)
        return carry

    lax.fori_loop(0, _CB, fill, 0)

    @pl.when(s == 0)
    def _():
        pltpu.sync_copy(zeros_hbm, acc_sh)

    plsc.subcore_barrier()

    def chunk(j, carry):
        base = pl.multiple_of(wid * _EPT + j * _CB, 8)
        pltpu.sync_copy(dst_hbm.at[pl.ds(base, _CB)], idx_v)
        pltpu.sync_copy(ones_v, acc_sh.at[idx_v], add=True)
        return carry

    lax.fori_loop(0, _EPT // _CB, chunk, 0)
    plsc.subcore_barrier()

    @pl.when(s < 10)
    def _():
        rbase = pl.multiple_of(s * 1000, 8)
        obase = pl.multiple_of(c * _N + s * 1000, 8)
        pltpu.sync_copy(acc_sh.at[pl.ds(rbase, 1000)],
                        out_hbm.at[pl.ds(obase, 1000)])


def _sc_degree(dst, zeros_n16):
    k = pl.kernel(
        _deg_body,
        out_type=jax.ShapeDtypeStruct((_NC * _N, 16), jnp.float32),
        mesh=_sc_mesh,
        scratch_types=[
            pltpu.VMEM((_CB,), jnp.int32),
            pltpu.VMEM((_CB, 16), jnp.float32),
            pltpu.VMEM_SHARED((_N, 16), jnp.float32),
            pltpu.SemaphoreType.DMA,
        ],
        compiler_params=pltpu.CompilerParams(use_tc_tiling_on_sc=False),
    )
    return k(dst, zeros_n16)


def _agg_body(g_hbm, src_hbm, dst_hbm, zeros_hbm, out_hbm,
              sidx, didx, rows_v, acc_sh, sem):
    c = lax.axis_index("c")
    s = lax.axis_index("s")
    wid = c * _NS + s

    @pl.when(s == 0)
    def _():
        pltpu.sync_copy(zeros_hbm, acc_sh)

    plsc.subcore_barrier()

    def chunk(j, carry):
        base = pl.multiple_of(wid * _EPT + j * _CB, 8)
        pltpu.sync_copy(src_hbm.at[pl.ds(base, _CB)], sidx)
        pltpu.sync_copy(dst_hbm.at[pl.ds(base, _CB)], didx)
        pltpu.async_copy(g_hbm.at[sidx], rows_v, sem).wait()
        pltpu.sync_copy(rows_v, acc_sh.at[didx], add=True)
        return carry

    lax.fori_loop(0, _EPT // _CB, chunk, 0)
    plsc.subcore_barrier()

    @pl.when(s < 10)
    def _():
        rbase = pl.multiple_of(s * 1000, 8)
        obase = pl.multiple_of(c * _N + s * 1000, 8)
        pltpu.sync_copy(acc_sh.at[pl.ds(rbase, 1000)],
                        out_hbm.at[pl.ds(obase, 1000)])


def _sc_edge_agg(g, src, dst, zeros_n32):
    k = pl.kernel(
        _agg_body,
        out_type=jax.ShapeDtypeStruct((_NC * _N, 32), jnp.float32),
        mesh=_sc_mesh,
        scratch_types=[
            pltpu.VMEM((_CB,), jnp.int32),
            pltpu.VMEM((_CB,), jnp.int32),
            pltpu.VMEM((_CB, 32), jnp.float32),
            pltpu.VMEM_SHARED((_N, 32), jnp.float32),
            pltpu.SemaphoreType.DMA,
        ],
        compiler_params=pltpu.CompilerParams(use_tc_tiling_on_sc=False),
    )
    return k(g, src, dst, zeros_n32)


def _dec_body(uc_hbm, un_hbm, src_hbm, dst_hbm, ewc_hbm, ewn_hbm,
              sidx, didx, a_v, b_v, oc_v, on_v, sem):
    c = lax.axis_index("c")
    s = lax.axis_index("s")
    wid = c * _NS + s

    def chunk(j, carry):
        base = pl.multiple_of(wid * _EPT + j * _CD, 8)
        pltpu.sync_copy(src_hbm.at[pl.ds(base, _CD)], sidx)
        pltpu.sync_copy(dst_hbm.at[pl.ds(base, _CD)], didx)

        for u_hbm, out_v in ((uc_hbm, oc_v), (un_hbm, on_v)):
            pltpu.async_copy(u_hbm.at[sidx], a_v, sem).wait()
            pltpu.async_copy(u_hbm.at[didx], b_v, sem).wait()

            def group(g, carry2):
                rowi = lax.iota(jnp.int32, 16) + g * 16
                acc = jnp.zeros((16,), jnp.float32)
                for kcol in range(10):
                    colk = jnp.full((16,), kcol, jnp.int32)
                    av = plsc.load_gather(a_v, [rowi, colk])
                    bv = plsc.load_gather(b_v, [rowi, colk])
                    acc = acc + av * bv
                out_v[pl.ds(g * 16, 16)] = 1.0 / (1.0 + jnp.exp(-acc))
                return carry2

            lax.fori_loop(0, _CD // 16, group, 0)

        pltpu.sync_copy(oc_v, ewc_hbm.at[pl.ds(base, _CD)])
        pltpu.sync_copy(on_v, ewn_hbm.at[pl.ds(base, _CD)])
        return carry

    lax.fori_loop(0, _EPT // _CD, chunk, 0)


def _sc_decoder(uc, un, src, dst):
    k = pl.kernel(
        _dec_body,
        out_type=(jax.ShapeDtypeStruct((_E,), jnp.float32),
                  jax.ShapeDtypeStruct((_E,), jnp.float32)),
        mesh=_sc_mesh,
        scratch_types=[
            pltpu.VMEM((_CD,), jnp.int32),
            pltpu.VMEM((_CD,), jnp.int32),
            pltpu.VMEM((_CD, 16), jnp.float32),
            pltpu.VMEM((_CD, 16), jnp.float32),
            pltpu.VMEM((_CD,), jnp.float32),
            pltpu.VMEM((_CD,), jnp.float32),
            pltpu.SemaphoreType.DMA,
        ],
        compiler_params=pltpu.CompilerParams(use_tc_tiling_on_sc=False,
                                             needs_layout_passes=False),
    )
    return k(uc, un, src, dst)


_BLK = 1000
_GRID = _N // _BLK


def _row_spec(d):
    return pl.BlockSpec((_BLK, d), lambda i: (i, 0))


def _full_spec(r, ccol):
    return pl.BlockSpec((r, ccol), lambda i: (0, 0))


def _tc12_body(x_ref, w_ref, d0_ref, d1_ref, h0_ref, dinv_ref, g0_ref):
    deg = d0_ref[...] + d1_ref[...] + 1.0
    dinv = 1.0 / jnp.sqrt(deg)
    h0 = jnp.dot(x_ref[...], w_ref[...], preferred_element_type=jnp.float32)
    h0_ref[...] = h0
    dinv_ref[...] = dinv
    g0_ref[...] = h0 * dinv


def _tc12(x, w, d0, d1):
    return pl.pallas_call(
        _tc12_body,
        grid=(_GRID,),
        in_specs=[_row_spec(128), _full_spec(128, 32), _row_spec(1), _row_spec(1)],
        out_specs=[_row_spec(32), _row_spec(1), _row_spec(32)],
        out_shape=[jax.ShapeDtypeStruct((_N, 32), jnp.float32),
                   jax.ShapeDtypeStruct((_N, 1), jnp.float32),
                   jax.ShapeDtypeStruct((_N, 32), jnp.float32)],
    )(x, w, d0, d1)


def _tc3_body(p0_ref, p1_ref, h0_ref, dinv_ref, b_ref, h_ref, g1_ref):
    dinv = dinv_ref[...]
    agg = dinv * (p0_ref[...] + p1_ref[...]) + dinv * dinv * h0_ref[...]
    h = jnp.maximum(agg + b_ref[...], 0.0)
    h_ref[...] = h
    g1_ref[...] = dinv * h


def _tc3(p0, p1, h0, dinv, b):
    return pl.pallas_call(
        _tc3_body,
        grid=(_GRID,),
        in_specs=[_row_spec(32), _row_spec(32), _row_spec(32), _row_spec(1),
                  _full_spec(1, 32)],
        out_specs=[_row_spec(32), _row_spec(32)],
        out_shape=[jax.ShapeDtypeStruct((_N, 32), jnp.float32),
                   jax.ShapeDtypeStruct((_N, 32), jnp.float32)],
    )(p0, p1, h0, dinv, b)


def _tc4_body(p0_ref, p1_ref, h_ref, dinv_ref,
              wmc_ref, bmc_ref, wmn_ref, bmn_ref,
              wlc_ref, blc_ref, wln_ref, bln_ref,
              ec_ref, en_ref,
              cw1_ref, cb1_ref, cw2_ref, cb2_ref,
              nw1_ref, nb1_ref, nw2_ref, nb2_ref,
              muc_ref, mun_ref, lvc_ref, lvn_ref, ucp_ref, unp_ref):
    dinv = dinv_ref[...]
    agg = dinv * (p0_ref[...] + p1_ref[...]) + dinv * dinv * h_ref[...]

    def mm(a, w_ref, b_ref):
        return jnp.dot(a, w_ref[...], preferred_element_type=jnp.float32) + b_ref[...]

    muc = mm(agg, wmc_ref, bmc_ref)
    mun = mm(agg, wmn_ref, bmn_ref)
    lvc = mm(agg, wlc_ref, blc_ref)
    lvn = mm(agg, wln_ref, bln_ref)
    muc_ref[...] = muc
    mun_ref[...] = mun
    lvc_ref[...] = lvc
    lvn_ref[...] = lvn
    zc = muc + ec_ref[...] * jnp.exp(0.5 * lvc)
    zn = mun + en_ref[...] * jnp.exp(0.5 * lvn)
    tc = jnp.tanh(mm(zc, cw1_ref, cb1_ref))
    tn = jnp.tanh(mm(zn, nw1_ref, nb1_ref))
    ucp_ref[...] = jnp.tanh(mm(tc, cw2_ref, cb2_ref))
    unp_ref[...] = jnp.tanh(mm(tn, nw2_ref, nb2_ref))


def _tc4(p0, p1, h, dinv, wmc, bmc, wmn, bmn, wlc, blc, wln, bln,
         ec, en, cw1, cb1, cw2, cb2, nw1, nb1, nw2, nb2):
    return pl.pallas_call(
        _tc4_body,
        grid=(_GRID,),
        in_specs=[_row_spec(32), _row_spec(32), _row_spec(32), _row_spec(1),
                  _full_spec(32, 64), _full_spec(1, 64),
                  _full_spec(32, 64), _full_spec(1, 64),
                  _full_spec(32, 64), _full_spec(1, 64),
                  _full_spec(32, 64), _full_spec(1, 64),
                  _row_spec(64), _row_spec(64),
                  _full_spec(64, 32), _full_spec(1, 32),
                  _full_spec(32, 16), _full_spec(1, 16),
                  _full_spec(64, 32), _full_spec(1, 32),
                  _full_spec(32, 16), _full_spec(1, 16)],
        out_specs=[_row_spec(64), _row_spec(64), _row_spec(64), _row_spec(64),
                   _row_spec(16), _row_spec(16)],
        out_shape=[jax.ShapeDtypeStruct((_N, 64), jnp.float32),
                   jax.ShapeDtypeStruct((_N, 64), jnp.float32),
                   jax.ShapeDtypeStruct((_N, 64), jnp.float32),
                   jax.ShapeDtypeStruct((_N, 64), jnp.float32),
                   jax.ShapeDtypeStruct((_N, 16), jnp.float32),
                   jax.ShapeDtypeStruct((_N, 16), jnp.float32)],
    )(p0, p1, h, dinv, wmc, bmc, wmn, bmn, wlc, blc, wln, bln,
      ec, en, cw1, cb1, cw2, cb2, nw1, nb1, nw2, nb2)


def kernel(x, edge_index, W_shared, b_shared, W_mu_c, b_mu_c, W_mu_nc, b_mu_nc,
           W_lv_c, b_lv_c, W_lv_nc, b_lv_nc,
           dc_fcW, dc_fcb, dc_fc2W, dc_fc2b,
           dn_fcW, dn_fcb, dn_fc2W, dn_fc2b):
    src, dst = edge_index[0], edge_index[1]

    zeros_n16 = jnp.zeros((_N, 16), jnp.float32)
    zeros_n32 = jnp.zeros((_N, 32), jnp.float32)

    degp = _sc_degree(dst, zeros_n16)
    d0 = degp[:_N, 0:1]
    d1 = degp[_N:, 0:1]

    h0, dinv, g0 = _tc12(x, W_shared, d0, d1)

    agg1p = _sc_edge_agg(g0, src, dst, zeros_n32)
    h, g1 = _tc3(agg1p[:_N], agg1p[_N:], h0, dinv, b_shared.reshape(1, 32))

    agg2p = _sc_edge_agg(g1, src, dst, zeros_n32)

    eps_c = jax.random.normal(jax.random.key(7), (_N, 64), jnp.float32)
    eps_nc = jax.random.normal(jax.random.key(8), (_N, 64), jnp.float32)

    pad_w = lambda w: jnp.pad(w, ((0, 0), (0, 6)))
    pad_b = lambda b: jnp.pad(b, ((0, 6),)).reshape(1, 16)

    mu_c, mu_nc, lv_c, lv_nc, ucp, unp = _tc4(
        agg2p[:_N], agg2p[_N:], h, dinv,
        W_mu_c, b_mu_c.reshape(1, 64), W_mu_nc, b_mu_nc.reshape(1, 64),
        W_lv_c, b_lv_c.reshape(1, 64), W_lv_nc, b_lv_nc.reshape(1, 64),
        eps_c, eps_nc,
        dc_fcW, dc_fcb.reshape(1, 32), pad_w(dc_fc2W), pad_b(dc_fc2b),
        dn_fcW, dn_fcb.reshape(1, 32), pad_w(dn_fc2W), pad_b(dn_fc2b))

    ew_c, ew_nc = _sc_decoder(ucp, unp, src, dst)
    return (ew_c, ew_nc, mu_c, mu_nc, lv_c, lv_nc)


# speedup vs baseline: 42.7620x; 1.1566x over previous
import functools

import jax
import jax.numpy as jnp
from jax import lax
from jax.experimental import pallas as pl
from jax.experimental.pallas import tpu as pltpu
from jax.experimental.pallas import tpu_sc as plsc

_N = 10000
_E = 320000
_NC = 2
_NS = 16
_NW = _NC * _NS
_EPT = _E // _NW
_CB = 1000
_CD = 400
_ROWS_PER_TILE = _N // _NS

_sc_mesh = plsc.VectorSubcoreMesh(core_axis_name="c", subcore_axis_name="s")


def _deg_body(dst_hbm, zeros_hbm, out_hbm, idx_v, ones_v, acc_sh, sem):
    c = lax.axis_index("c")
    s = lax.axis_index("s")
    wid = c * _NS + s

    def fill(i, carry):
        ones_v[i, :] = jnp.full((16,), 1.0, jnp.float32)
        return carry

    lax.fori_loop(0, _CB, fill, 0)

    @pl.when(s == 0)
    def _():
        pltpu.sync_copy(zeros_hbm, acc_sh)

    plsc.subcore_barrier()

    def chunk(j, carry):
        base = pl.multiple_of(wid * _EPT + j * _CB, 8)
        pltpu.sync_copy(dst_hbm.at[pl.ds(base, _CB)], idx_v)
        pltpu.sync_copy(ones_v, acc_sh.at[idx_v], add=True)
        return carry

    lax.fori_loop(0, _EPT // _CB, chunk, 0)
    plsc.subcore_barrier()

    @pl.when(s < 10)
    def _():
        rbase = pl.multiple_of(s * 1000, 8)
        obase = pl.multiple_of(c * _N + s * 1000, 8)
        pltpu.sync_copy(acc_sh.at[pl.ds(rbase, 1000)],
                        out_hbm.at[pl.ds(obase, 1000)])


def _sc_degree(dst, zeros_n16):
    k = pl.kernel(
        _deg_body,
        out_type=jax.ShapeDtypeStruct((_NC * _N, 16), jnp.float32),
        mesh=_sc_mesh,
        scratch_types=[
            pltpu.VMEM((_CB,), jnp.int32),
            pltpu.VMEM((_CB, 16), jnp.float32),
            pltpu.VMEM_SHARED((_N, 16), jnp.float32),
            pltpu.SemaphoreType.DMA,
        ],
        compiler_params=pltpu.CompilerParams(use_tc_tiling_on_sc=False),
    )
    return k(dst, zeros_n16)


def _agg_body(g_hbm, src_hbm, dst_hbm, zeros_hbm, out_hbm,
              sidx0, sidx1, didx0, didx1, rows0, rows1, acc_sh, sem0, sem1):
    c = lax.axis_index("c")
    s = lax.axis_index("s")
    wid = c * _NS + s

    @pl.when(s == 0)
    def _():
        pltpu.sync_copy(zeros_hbm, acc_sh)

    plsc.subcore_barrier()

    nchunks = _EPT // _CB
    sidx = (sidx0, sidx1)
    didx = (didx0, didx1)
    rows = (rows0, rows1)
    sems = (sem0, sem1)

    def load_and_fire(j, b):
        base = pl.multiple_of(wid * _EPT + j * _CB, 8)
        pltpu.sync_copy(src_hbm.at[pl.ds(base, _CB)], sidx[b])
        pltpu.sync_copy(dst_hbm.at[pl.ds(base, _CB)], didx[b])
        return pltpu.async_copy(g_hbm.at[sidx[b]], rows[b], sems[b])

    pending = load_and_fire(0, 0)
    for j in range(nchunks):
        b = j % 2
        pending.wait()
        if j + 1 < nchunks:
            pending = load_and_fire(j + 1, (j + 1) % 2)
        pltpu.sync_copy(rows[b], acc_sh.at[didx[b]], add=True)

    plsc.subcore_barrier()

    @pl.when(s < 10)
    def _():
        rbase = pl.multiple_of(s * 1000, 8)
        obase = pl.multiple_of(c * _N + s * 1000, 8)
        pltpu.sync_copy(acc_sh.at[pl.ds(rbase, 1000)],
                        out_hbm.at[pl.ds(obase, 1000)])


def _sc_edge_agg(g, src, dst, zeros_n32):
    k = pl.kernel(
        _agg_body,
        out_type=jax.ShapeDtypeStruct((_NC * _N, 32), jnp.float32),
        mesh=_sc_mesh,
        scratch_types=[
            pltpu.VMEM((_CB,), jnp.int32),
            pltpu.VMEM((_CB,), jnp.int32),
            pltpu.VMEM((_CB,), jnp.int32),
            pltpu.VMEM((_CB,), jnp.int32),
            pltpu.VMEM((_CB, 32), jnp.float32),
            pltpu.VMEM((_CB, 32), jnp.float32),
            pltpu.VMEM_SHARED((_N, 32), jnp.float32),
            pltpu.SemaphoreType.DMA,
            pltpu.SemaphoreType.DMA,
        ],
        compiler_params=pltpu.CompilerParams(use_tc_tiling_on_sc=False),
    )
    return k(g, src, dst, zeros_n32)


def _dec_body(uc_hbm, un_hbm, src_hbm, dst_hbm, ewc_hbm, ewn_hbm,
              sidx_all, didx_all,
              ac0, ac1, bc0, bc1, an0, an1, bn0, bn1,
              oc0, oc1, on0, on1,
              gsem0, gsem1, osem0, osem1):
    c = lax.axis_index("c")
    s = lax.axis_index("s")
    wid = c * _NS + s
    ebase = pl.multiple_of(wid * _EPT, 8)

    pltpu.sync_copy(src_hbm.at[pl.ds(ebase, _EPT)], sidx_all)
    pltpu.sync_copy(dst_hbm.at[pl.ds(ebase, _EPT)], didx_all)

    nchunks = _EPT // _CD
    acb = (ac0, ac1)
    bcb = (bc0, bc1)
    anb = (an0, an1)
    bnb = (bn0, bn1)
    ocb = (oc0, oc1)
    onb = (on0, on1)
    gsem = (gsem0, gsem1)
    osem = (osem0, osem1)

    def fire(j, b):
        si = sidx_all.at[pl.ds(j * _CD, _CD)]
        di = didx_all.at[pl.ds(j * _CD, _CD)]
        return (pltpu.async_copy(uc_hbm.at[si], acb[b], gsem[b]),
                pltpu.async_copy(uc_hbm.at[di], bcb[b], gsem[b]),
                pltpu.async_copy(un_hbm.at[si], anb[b], gsem[b]),
                pltpu.async_copy(un_hbm.at[di], bnb[b], gsem[b]))

    pending = fire(0, 0)
    out_pending = [None, None]
    for j in range(nchunks):
        b = j % 2
        for d in pending:
            d.wait()
        if j + 1 < nchunks:
            pending = fire(j + 1, (j + 1) % 2)
        if out_pending[b] is not None:
            for d in out_pending[b]:
                d.wait()

        for a_v, b_v, out_v in ((acb[b], bcb[b], ocb[b]),
                                (anb[b], bnb[b], onb[b])):
            def group(g, carry2, a_v=a_v, b_v=b_v, out_v=out_v):
                rowi = lax.iota(jnp.int32, 16) + g * 16
                acc = jnp.zeros((16,), jnp.float32)
                for kcol in range(10):
                    colk = jnp.full((16,), kcol, jnp.int32)
                    av = plsc.load_gather(a_v, [rowi, colk])
                    bv = plsc.load_gather(b_v, [rowi, colk])
                    acc = acc + av * bv
                out_v[pl.ds(g * 16, 16)] = 1.0 / (1.0 + jnp.exp(-acc))
                return carry2

            lax.fori_loop(0, _CD // 16, group, 0)

        obase = pl.multiple_of(ebase + j * _CD, 8)
        out_pending[b] = (
            pltpu.async_copy(ocb[b], ewc_hbm.at[pl.ds(obase, _CD)], osem[b]),
            pltpu.async_copy(onb[b], ewn_hbm.at[pl.ds(obase, _CD)], osem[b]))

    for b in range(2):
        if out_pending[b] is not None:
            for d in out_pending[b]:
                d.wait()


def _sc_decoder(uc, un, src, dst):
    k = pl.kernel(
        _dec_body,
        out_type=(jax.ShapeDtypeStruct((_E,), jnp.float32),
                  jax.ShapeDtypeStruct((_E,), jnp.float32)),
        mesh=_sc_mesh,
        scratch_types=[
            pltpu.VMEM((_EPT,), jnp.int32),
            pltpu.VMEM((_EPT,), jnp.int32),
            pltpu.VMEM((_CD, 16), jnp.float32),
            pltpu.VMEM((_CD, 16), jnp.float32),
            pltpu.VMEM((_CD, 16), jnp.float32),
            pltpu.VMEM((_CD, 16), jnp.float32),
            pltpu.VMEM((_CD, 16), jnp.float32),
            pltpu.VMEM((_CD, 16), jnp.float32),
            pltpu.VMEM((_CD, 16), jnp.float32),
            pltpu.VMEM((_CD, 16), jnp.float32),
            pltpu.VMEM((_CD,), jnp.float32),
            pltpu.VMEM((_CD,), jnp.float32),
            pltpu.VMEM((_CD,), jnp.float32),
            pltpu.VMEM((_CD,), jnp.float32),
            pltpu.SemaphoreType.DMA,
            pltpu.SemaphoreType.DMA,
            pltpu.SemaphoreType.DMA,
            pltpu.SemaphoreType.DMA,
        ],
        compiler_params=pltpu.CompilerParams(use_tc_tiling_on_sc=False,
                                             needs_layout_passes=False),
    )
    return k(uc, un, src, dst)


_BLK = 1000
_GRID = _N // _BLK


def _row_spec(d):
    return pl.BlockSpec((_BLK, d), lambda i: (i, 0))


def _full_spec(r, ccol):
    return pl.BlockSpec((r, ccol), lambda i: (0, 0))


def _tc12_body(x_ref, w_ref, d0_ref, d1_ref, h0_ref, dinv_ref, g0_ref):
    deg = d0_ref[...] + d1_ref[...] + 1.0
    dinv = 1.0 / jnp.sqrt(deg)
    h0 = jnp.dot(x_ref[...], w_ref[...], preferred_element_type=jnp.float32)
    h0_ref[...] = h0
    dinv_ref[...] = dinv
    g0_ref[...] = h0 * dinv


def _tc12(x, w, d0, d1):
    return pl.pallas_call(
        _tc12_body,
        grid=(_GRID,),
        in_specs=[_row_spec(128), _full_spec(128, 32), _row_spec(1), _row_spec(1)],
        out_specs=[_row_spec(32), _row_spec(1), _row_spec(32)],
        out_shape=[jax.ShapeDtypeStruct((_N, 32), jnp.float32),
                   jax.ShapeDtypeStruct((_N, 1), jnp.float32),
                   jax.ShapeDtypeStruct((_N, 32), jnp.float32)],
    )(x, w, d0, d1)


def _tc3_body(p0_ref, p1_ref, h0_ref, dinv_ref, b_ref, h_ref, g1_ref):
    dinv = dinv_ref[...]
    agg = dinv * (p0_ref[...] + p1_ref[...]) + dinv * dinv * h0_ref[...]
    h = jnp.maximum(agg + b_ref[...], 0.0)
    h_ref[...] = h
    g1_ref[...] = dinv * h


def _tc3(p0, p1, h0, dinv, b):
    return pl.pallas_call(
        _tc3_body,
        grid=(_GRID,),
        in_specs=[_row_spec(32), _row_spec(32), _row_spec(32), _row_spec(1),
                  _full_spec(1, 32)],
        out_specs=[_row_spec(32), _row_spec(32)],
        out_shape=[jax.ShapeDtypeStruct((_N, 32), jnp.float32),
                   jax.ShapeDtypeStruct((_N, 32), jnp.float32)],
    )(p0, p1, h0, dinv, b)


def _tc4_body(p0_ref, p1_ref, h_ref, dinv_ref,
              wmc_ref, bmc_ref, wmn_ref, bmn_ref,
              wlc_ref, blc_ref, wln_ref, bln_ref,
              ec_ref, en_ref,
              cw1_ref, cb1_ref, cw2_ref, cb2_ref,
              nw1_ref, nb1_ref, nw2_ref, nb2_ref,
              muc_ref, mun_ref, lvc_ref, lvn_ref, ucp_ref, unp_ref):
    dinv = dinv_ref[...]
    agg = dinv * (p0_ref[...] + p1_ref[...]) + dinv * dinv * h_ref[...]

    def mm(a, w_ref, b_ref):
        return jnp.dot(a, w_ref[...], preferred_element_type=jnp.float32) + b_ref[...]

    muc = mm(agg, wmc_ref, bmc_ref)
    mun = mm(agg, wmn_ref, bmn_ref)
    lvc = mm(agg, wlc_ref, blc_ref)
    lvn = mm(agg, wln_ref, bln_ref)
    muc_ref[...] = muc
    mun_ref[...] = mun
    lvc_ref[...] = lvc
    lvn_ref[...] = lvn
    zc = muc + ec_ref[...] * jnp.exp(0.5 * lvc)
    zn = mun + en_ref[...] * jnp.exp(0.5 * lvn)
    tc = jnp.tanh(mm(zc, cw1_ref, cb1_ref))
    tn = jnp.tanh(mm(zn, nw1_ref, nb1_ref))
    ucp_ref[...] = jnp.tanh(mm(tc, cw2_ref, cb2_ref))
    unp_ref[...] = jnp.tanh(mm(tn, nw2_ref, nb2_ref))


def _tc4(p0, p1, h, dinv, wmc, bmc, wmn, bmn, wlc, blc, wln, bln,
         ec, en, cw1, cb1, cw2, cb2, nw1, nb1, nw2, nb2):
    return pl.pallas_call(
        _tc4_body,
        grid=(_GRID,),
        in_specs=[_row_spec(32), _row_spec(32), _row_spec(32), _row_spec(1),
                  _full_spec(32, 64), _full_spec(1, 64),
                  _full_spec(32, 64), _full_spec(1, 64),
                  _full_spec(32, 64), _full_spec(1, 64),
                  _full_spec(32, 64), _full_spec(1, 64),
                  _row_spec(64), _row_spec(64),
                  _full_spec(64, 32), _full_spec(1, 32),
                  _full_spec(32, 16), _full_spec(1, 16),
                  _full_spec(64, 32), _full_spec(1, 32),
                  _full_spec(32, 16), _full_spec(1, 16)],
        out_specs=[_row_spec(64), _row_spec(64), _row_spec(64), _row_spec(64),
                   _row_spec(16), _row_spec(16)],
        out_shape=[jax.ShapeDtypeStruct((_N, 64), jnp.float32),
                   jax.ShapeDtypeStruct((_N, 64), jnp.float32),
                   jax.ShapeDtypeStruct((_N, 64), jnp.float32),
                   jax.ShapeDtypeStruct((_N, 64), jnp.float32),
                   jax.ShapeDtypeStruct((_N, 16), jnp.float32),
                   jax.ShapeDtypeStruct((_N, 16), jnp.float32)],
    )(p0, p1, h, dinv, wmc, bmc, wmn, bmn, wlc, blc, wln, bln,
      ec, en, cw1, cb1, cw2, cb2, nw1, nb1, nw2, nb2)


def kernel(x, edge_index, W_shared, b_shared, W_mu_c, b_mu_c, W_mu_nc, b_mu_nc,
           W_lv_c, b_lv_c, W_lv_nc, b_lv_nc,
           dc_fcW, dc_fcb, dc_fc2W, dc_fc2b,
           dn_fcW, dn_fcb, dn_fc2W, dn_fc2b):
    src, dst = edge_index[0], edge_index[1]

    zeros_n16 = jnp.zeros((_N, 16), jnp.float32)
    zeros_n32 = jnp.zeros((_N, 32), jnp.float32)

    degp = _sc_degree(dst, zeros_n16)
    d0 = degp[:_N, 0:1]
    d1 = degp[_N:, 0:1]

    h0, dinv, g0 = _tc12(x, W_shared, d0, d1)

    agg1p = _sc_edge_agg(g0, src, dst, zeros_n32)
    h, g1 = _tc3(agg1p[:_N], agg1p[_N:], h0, dinv, b_shared.reshape(1, 32))

    agg2p = _sc_edge_agg(g1, src, dst, zeros_n32)

    eps_c = jax.random.normal(jax.random.key(7), (_N, 64), jnp.float32)
    eps_nc = jax.random.normal(jax.random.key(8), (_N, 64), jnp.float32)

    pad_w = lambda w: jnp.pad(w, ((0, 0), (0, 6)))
    pad_b = lambda b: jnp.pad(b, ((0, 6),)).reshape(1, 16)

    mu_c, mu_nc, lv_c, lv_nc, ucp, unp = _tc4(
        agg2p[:_N], agg2p[_N:], h, dinv,
        W_mu_c, b_mu_c.reshape(1, 64), W_mu_nc, b_mu_nc.reshape(1, 64),
        W_lv_c, b_lv_c.reshape(1, 64), W_lv_nc, b_lv_nc.reshape(1, 64),
        eps_c, eps_nc,
        dc_fcW, dc_fcb.reshape(1, 32), pad_w(dc_fc2W), pad_b(dc_fc2b),
        dn_fcW, dn_fcb.reshape(1, 32), pad_w(dn_fc2W), pad_b(dn_fc2b))

    ew_c, ew_nc = _sc_decoder(ucp, unp, src, dst)
    return (ew_c, ew_nc, mu_c, mu_nc, lv_c, lv_nc)


# speedup vs baseline: 48.0022x; 1.1225x over previous
import functools

import jax
import jax.numpy as jnp
from jax import lax
from jax.experimental import pallas as pl
from jax.experimental.pallas import tpu as pltpu
from jax.experimental.pallas import tpu_sc as plsc

_N = 10000
_E = 320000
_NC = 2
_NS = 16
_NW = _NC * _NS
_EPT = _E // _NW
_CB = 1000
_CD = 2000
_ROWS_PER_TILE = _N // _NS

_sc_mesh = plsc.VectorSubcoreMesh(core_axis_name="c", subcore_axis_name="s")


def _deg_body(dst_hbm, zeros_hbm, out_hbm, idx_v, ones_v, acc_sh, sem):
    c = lax.axis_index("c")
    s = lax.axis_index("s")
    wid = c * _NS + s

    def fill(i, carry):
        ones_v[i, :] = jnp.full((16,), 1.0, jnp.float32)
        return carry

    lax.fori_loop(0, _CB, fill, 0)

    @pl.when(s == 0)
    def _():
        pltpu.sync_copy(zeros_hbm, acc_sh)

    plsc.subcore_barrier()

    def chunk(j, carry):
        base = pl.multiple_of(wid * _EPT + j * _CB, 8)
        pltpu.sync_copy(dst_hbm.at[pl.ds(base, _CB)], idx_v)
        pltpu.sync_copy(ones_v, acc_sh.at[idx_v], add=True)
        return carry

    lax.fori_loop(0, _EPT // _CB, chunk, 0)
    plsc.subcore_barrier()

    @pl.when(s < 10)
    def _():
        rbase = pl.multiple_of(s * 1000, 8)
        obase = pl.multiple_of(c * _N + s * 1000, 8)
        pltpu.sync_copy(acc_sh.at[pl.ds(rbase, 1000)],
                        out_hbm.at[pl.ds(obase, 1000)])


def _sc_degree(dst, zeros_n16):
    k = pl.kernel(
        _deg_body,
        out_type=jax.ShapeDtypeStruct((_NC * _N, 16), jnp.float32),
        mesh=_sc_mesh,
        scratch_types=[
            pltpu.VMEM((_CB,), jnp.int32),
            pltpu.VMEM((_CB, 16), jnp.float32),
            pltpu.VMEM_SHARED((_N, 16), jnp.float32),
            pltpu.SemaphoreType.DMA,
        ],
        compiler_params=pltpu.CompilerParams(use_tc_tiling_on_sc=False),
    )
    return k(dst, zeros_n16)


def _agg_body(g_hbm, src_hbm, dst_hbm, zeros_hbm, out_hbm,
              sidx0, sidx1, didx0, didx1, rows0, rows1, acc_sh, sem0, sem1):
    c = lax.axis_index("c")
    s = lax.axis_index("s")
    wid = c * _NS + s

    @pl.when(s == 0)
    def _():
        pltpu.sync_copy(zeros_hbm, acc_sh)

    plsc.subcore_barrier()

    nchunks = _EPT // _CB
    sidx = (sidx0, sidx1)
    didx = (didx0, didx1)
    rows = (rows0, rows1)
    sems = (sem0, sem1)

    def load_and_fire(j, b):
        base = pl.multiple_of(wid * _EPT + j * _CB, 8)
        pltpu.sync_copy(src_hbm.at[pl.ds(base, _CB)], sidx[b])
        pltpu.sync_copy(dst_hbm.at[pl.ds(base, _CB)], didx[b])
        return pltpu.async_copy(g_hbm.at[sidx[b]], rows[b], sems[b])

    pending = load_and_fire(0, 0)
    for j in range(nchunks):
        b = j % 2
        pending.wait()
        if j + 1 < nchunks:
            pending = load_and_fire(j + 1, (j + 1) % 2)
        pltpu.sync_copy(rows[b], acc_sh.at[didx[b]], add=True)

    plsc.subcore_barrier()

    @pl.when(s < 10)
    def _():
        rbase = pl.multiple_of(s * 1000, 8)
        obase = pl.multiple_of(c * _N + s * 1000, 8)
        pltpu.sync_copy(acc_sh.at[pl.ds(rbase, 1000)],
                        out_hbm.at[pl.ds(obase, 1000)])


def _sc_edge_agg(g, src, dst, zeros_n32):
    k = pl.kernel(
        _agg_body,
        out_type=jax.ShapeDtypeStruct((_NC * _N, 32), jnp.float32),
        mesh=_sc_mesh,
        scratch_types=[
            pltpu.VMEM((_CB,), jnp.int32),
            pltpu.VMEM((_CB,), jnp.int32),
            pltpu.VMEM((_CB,), jnp.int32),
            pltpu.VMEM((_CB,), jnp.int32),
            pltpu.VMEM((_CB, 32), jnp.float32),
            pltpu.VMEM((_CB, 32), jnp.float32),
            pltpu.VMEM_SHARED((_N, 32), jnp.float32),
            pltpu.SemaphoreType.DMA,
            pltpu.SemaphoreType.DMA,
        ],
        compiler_params=pltpu.CompilerParams(use_tc_tiling_on_sc=False),
    )
    return k(g, src, dst, zeros_n32)


def _dec_body(ucT_hbm, unT_hbm, src_hbm, dst_hbm, ewc_hbm, ewn_hbm,
              sidx_all, didx_all, uT_v, o0, o1, osem0, osem1):
    c = lax.axis_index("c")
    s = lax.axis_index("s")
    wid = c * _NS + s
    ebase = pl.multiple_of(wid * _EPT, 8)

    pltpu.sync_copy(src_hbm.at[pl.ds(ebase, _EPT)], sidx_all)
    pltpu.sync_copy(dst_hbm.at[pl.ds(ebase, _EPT)], didx_all)

    nchunks = _EPT // _CD
    outs = (o0, o1)
    osem = (osem0, osem1)
    out_pending = [None, None]

    for uT_hbm, ew_hbm in ((ucT_hbm, ewc_hbm), (unT_hbm, ewn_hbm)):
        pltpu.sync_copy(uT_hbm, uT_v)
        for j in range(nchunks):
            b = j % 2
            if out_pending[b] is not None:
                out_pending[b].wait()

            def group(g, carry, j=j, b=b):
                si = sidx_all[pl.ds(j * _CD + g * 16, 16)]
                di = didx_all[pl.ds(j * _CD + g * 16, 16)]
                acc = jnp.zeros((16,), jnp.float32)
                for kcol in range(10):
                    rowk = jnp.full((16,), kcol, jnp.int32)
                    av = plsc.load_gather(uT_v, [rowk, si])
                    bv = plsc.load_gather(uT_v, [rowk, di])
                    acc = acc + av * bv
                outs[b][pl.ds(g * 16, 16)] = 1.0 / (1.0 + jnp.exp(-acc))
                return carry

            lax.fori_loop(0, _CD // 16, group, 0)
            obase = pl.multiple_of(ebase + j * _CD, 8)
            out_pending[b] = pltpu.async_copy(
                outs[b], ew_hbm.at[pl.ds(obase, _CD)], osem[b])
        for b in range(2):
            if out_pending[b] is not None:
                out_pending[b].wait()
                out_pending[b] = None


def _sc_decoder(ucT, unT, src, dst):
    k = pl.kernel(
        _dec_body,
        out_type=(jax.ShapeDtypeStruct((_E,), jnp.float32),
                  jax.ShapeDtypeStruct((_E,), jnp.float32)),
        mesh=_sc_mesh,
        scratch_types=[
            pltpu.VMEM((_EPT,), jnp.int32),
            pltpu.VMEM((_EPT,), jnp.int32),
            pltpu.VMEM((10, _N), jnp.float32),
            pltpu.VMEM((_CD,), jnp.float32),
            pltpu.VMEM((_CD,), jnp.float32),
            pltpu.SemaphoreType.DMA,
            pltpu.SemaphoreType.DMA,
        ],
        compiler_params=pltpu.CompilerParams(use_tc_tiling_on_sc=False,
                                             needs_layout_passes=False),
    )
    return k(ucT, unT, src, dst)


_BLK = 1000
_GRID = _N // _BLK


def _row_spec(d):
    return pl.BlockSpec((_BLK, d), lambda i: (i, 0))


def _full_spec(r, ccol):
    return pl.BlockSpec((r, ccol), lambda i: (0, 0))


def _tc12_body(x_ref, w_ref, d0_ref, d1_ref, h0_ref, dinv_ref, g0_ref):
    deg = d0_ref[...] + d1_ref[...] + 1.0
    dinv = 1.0 / jnp.sqrt(deg)
    h0 = jnp.dot(x_ref[...], w_ref[...], preferred_element_type=jnp.float32)
    h0_ref[...] = h0
    dinv_ref[...] = dinv
    g0_ref[...] = h0 * dinv


def _tc12(x, w, d0, d1):
    return pl.pallas_call(
        _tc12_body,
        grid=(_GRID,),
        in_specs=[_row_spec(128), _full_spec(128, 32), _row_spec(1), _row_spec(1)],
        out_specs=[_row_spec(32), _row_spec(1), _row_spec(32)],
        out_shape=[jax.ShapeDtypeStruct((_N, 32), jnp.float32),
                   jax.ShapeDtypeStruct((_N, 1), jnp.float32),
                   jax.ShapeDtypeStruct((_N, 32), jnp.float32)],
    )(x, w, d0, d1)


def _tc3_body(p0_ref, p1_ref, h0_ref, dinv_ref, b_ref, h_ref, g1_ref):
    dinv = dinv_ref[...]
    agg = dinv * (p0_ref[...] + p1_ref[...]) + dinv * dinv * h0_ref[...]
    h = jnp.maximum(agg + b_ref[...], 0.0)
    h_ref[...] = h
    g1_ref[...] = dinv * h


def _tc3(p0, p1, h0, dinv, b):
    return pl.pallas_call(
        _tc3_body,
        grid=(_GRID,),
        in_specs=[_row_spec(32), _row_spec(32), _row_spec(32), _row_spec(1),
                  _full_spec(1, 32)],
        out_specs=[_row_spec(32), _row_spec(32)],
        out_shape=[jax.ShapeDtypeStruct((_N, 32), jnp.float32),
                   jax.ShapeDtypeStruct((_N, 32), jnp.float32)],
    )(p0, p1, h0, dinv, b)


def _tc4_body(p0_ref, p1_ref, h_ref, dinv_ref,
              wmc_ref, bmc_ref, wmn_ref, bmn_ref,
              wlc_ref, blc_ref, wln_ref, bln_ref,
              ec_ref, en_ref,
              cw1_ref, cb1_ref, cw2_ref, cb2_ref,
              nw1_ref, nb1_ref, nw2_ref, nb2_ref,
              muc_ref, mun_ref, lvc_ref, lvn_ref, ucp_ref, unp_ref):
    dinv = dinv_ref[...]
    agg = dinv * (p0_ref[...] + p1_ref[...]) + dinv * dinv * h_ref[...]

    def mm(a, w_ref, b_ref):
        return jnp.dot(a, w_ref[...], preferred_element_type=jnp.float32) + b_ref[...]

    muc = mm(agg, wmc_ref, bmc_ref)
    mun = mm(agg, wmn_ref, bmn_ref)
    lvc = mm(agg, wlc_ref, blc_ref)
    lvn = mm(agg, wln_ref, bln_ref)
    muc_ref[...] = muc
    mun_ref[...] = mun
    lvc_ref[...] = lvc
    lvn_ref[...] = lvn
    zc = muc + ec_ref[...] * jnp.exp(0.5 * lvc)
    zn = mun + en_ref[...] * jnp.exp(0.5 * lvn)
    tc = jnp.tanh(mm(zc, cw1_ref, cb1_ref))
    tn = jnp.tanh(mm(zn, nw1_ref, nb1_ref))
    ucp_ref[...] = jnp.tanh(mm(tc, cw2_ref, cb2_ref))
    unp_ref[...] = jnp.tanh(mm(tn, nw2_ref, nb2_ref))


def _tc4(p0, p1, h, dinv, wmc, bmc, wmn, bmn, wlc, blc, wln, bln,
         ec, en, cw1, cb1, cw2, cb2, nw1, nb1, nw2, nb2):
    return pl.pallas_call(
        _tc4_body,
        grid=(_GRID,),
        in_specs=[_row_spec(32), _row_spec(32), _row_spec(32), _row_spec(1),
                  _full_spec(32, 64), _full_spec(1, 64),
                  _full_spec(32, 64), _full_spec(1, 64),
                  _full_spec(32, 64), _full_spec(1, 64),
                  _full_spec(32, 64), _full_spec(1, 64),
                  _row_spec(64), _row_spec(64),
                  _full_spec(64, 32), _full_spec(1, 32),
                  _full_spec(32, 16), _full_spec(1, 16),
                  _full_spec(64, 32), _full_spec(1, 32),
                  _full_spec(32, 16), _full_spec(1, 16)],
        out_specs=[_row_spec(64), _row_spec(64), _row_spec(64), _row_spec(64),
                   _row_spec(16), _row_spec(16)],
        out_shape=[jax.ShapeDtypeStruct((_N, 64), jnp.float32),
                   jax.ShapeDtypeStruct((_N, 64), jnp.float32),
                   jax.ShapeDtypeStruct((_N, 64), jnp.float32),
                   jax.ShapeDtypeStruct((_N, 64), jnp.float32),
                   jax.ShapeDtypeStruct((_N, 16), jnp.float32),
                   jax.ShapeDtypeStruct((_N, 16), jnp.float32)],
    )(p0, p1, h, dinv, wmc, bmc, wmn, bmn, wlc, blc, wln, bln,
      ec, en, cw1, cb1, cw2, cb2, nw1, nb1, nw2, nb2)


def kernel(x, edge_index, W_shared, b_shared, W_mu_c, b_mu_c, W_mu_nc, b_mu_nc,
           W_lv_c, b_lv_c, W_lv_nc, b_lv_nc,
           dc_fcW, dc_fcb, dc_fc2W, dc_fc2b,
           dn_fcW, dn_fcb, dn_fc2W, dn_fc2b):
    src, dst = edge_index[0], edge_index[1]

    zeros_n16 = jnp.zeros((_N, 16), jnp.float32)
    zeros_n32 = jnp.zeros((_N, 32), jnp.float32)

    degp = _sc_degree(dst, zeros_n16)
    d0 = degp[:_N, 0:1]
    d1 = degp[_N:, 0:1]

    h0, dinv, g0 = _tc12(x, W_shared, d0, d1)

    agg1p = _sc_edge_agg(g0, src, dst, zeros_n32)
    h, g1 = _tc3(agg1p[:_N], agg1p[_N:], h0, dinv, b_shared.reshape(1, 32))

    agg2p = _sc_edge_agg(g1, src, dst, zeros_n32)

    eps_c = jax.random.normal(jax.random.key(7), (_N, 64), jnp.float32)
    eps_nc = jax.random.normal(jax.random.key(8), (_N, 64), jnp.float32)

    pad_w = lambda w: jnp.pad(w, ((0, 0), (0, 6)))
    pad_b = lambda b: jnp.pad(b, ((0, 6),)).reshape(1, 16)

    mu_c, mu_nc, lv_c, lv_nc, ucp, unp = _tc4(
        agg2p[:_N], agg2p[_N:], h, dinv,
        W_mu_c, b_mu_c.reshape(1, 64), W_mu_nc, b_mu_nc.reshape(1, 64),
        W_lv_c, b_lv_c.reshape(1, 64), W_lv_nc, b_lv_nc.reshape(1, 64),
        eps_c, eps_nc,
        dc_fcW, dc_fcb.reshape(1, 32), pad_w(dc_fc2W), pad_b(dc_fc2b),
        dn_fcW, dn_fcb.reshape(1, 32), pad_w(dn_fc2W), pad_b(dn_fc2b))

    ew_c, ew_nc = _sc_decoder(jnp.transpose(ucp[:, :10]),
                              jnp.transpose(unp[:, :10]), src, dst)
    return (ew_c, ew_nc, mu_c, mu_nc, lv_c, lv_nc)


# speedup vs baseline: 50.9068x; 1.0605x over previous
import functools

import jax
import jax.numpy as jnp
from jax import lax
from jax.experimental import pallas as pl
from jax.experimental.pallas import tpu as pltpu
from jax.experimental.pallas import tpu_sc as plsc

_N = 10000
_E = 320000
_NC = 2
_NS = 16
_NW = _NC * _NS
_EPT = _E // _NW
_CB = 1000
_CD = 2000
_ROWS_PER_TILE = _N // _NS

_sc_mesh = plsc.VectorSubcoreMesh(core_axis_name="c", subcore_axis_name="s")


def _deg_body(dst_hbm, zeros_hbm, out_hbm, idx_v, ones_v, acc_sh, sem):
    c = lax.axis_index("c")
    s = lax.axis_index("s")
    wid = c * _NS + s

    def fill(i, carry):
        ones_v[i, :] = jnp.full((16,), 1.0, jnp.float32)
        return carry

    lax.fori_loop(0, _CB, fill, 0)

    @pl.when(s == 0)
    def _():
        pltpu.sync_copy(zeros_hbm, acc_sh)

    plsc.subcore_barrier()

    def chunk(j, carry):
        base = pl.multiple_of(wid * _EPT + j * _CB, 8)
        pltpu.sync_copy(dst_hbm.at[pl.ds(base, _CB)], idx_v)
        pltpu.sync_copy(ones_v, acc_sh.at[idx_v], add=True)
        return carry

    lax.fori_loop(0, _EPT // _CB, chunk, 0)
    plsc.subcore_barrier()

    @pl.when(s < 10)
    def _():
        rbase = pl.multiple_of(s * 1000, 8)
        obase = pl.multiple_of(c * _N + s * 1000, 8)
        pltpu.sync_copy(acc_sh.at[pl.ds(rbase, 1000)],
                        out_hbm.at[pl.ds(obase, 1000)])


def _sc_degree(dst, zeros_n16):
    k = pl.kernel(
        _deg_body,
        out_type=jax.ShapeDtypeStruct((_NC * _N, 16), jnp.float32),
        mesh=_sc_mesh,
        scratch_types=[
            pltpu.VMEM((_CB,), jnp.int32),
            pltpu.VMEM((_CB, 16), jnp.float32),
            pltpu.VMEM_SHARED((_N, 16), jnp.float32),
            pltpu.SemaphoreType.DMA,
        ],
        compiler_params=pltpu.CompilerParams(use_tc_tiling_on_sc=False),
    )
    return k(dst, zeros_n16)


def _agg_body(g_hbm, src_hbm, dst_hbm, zeros_hbm, out_hbm,
              sidx0, sidx1, didx0, didx1, rows0, rows1, acc_sh, sem0, sem1):
    c = lax.axis_index("c")
    s = lax.axis_index("s")
    wid = c * _NS + s

    @pl.when(s == 0)
    def _():
        pltpu.sync_copy(zeros_hbm, acc_sh)

    plsc.subcore_barrier()

    nchunks = _EPT // _CB
    sidx = (sidx0, sidx1)
    didx = (didx0, didx1)
    rows = (rows0, rows1)
    sems = (sem0, sem1)

    def load_and_fire(j, b):
        base = pl.multiple_of(wid * _EPT + j * _CB, 8)
        pltpu.sync_copy(src_hbm.at[pl.ds(base, _CB)], sidx[b])
        pltpu.sync_copy(dst_hbm.at[pl.ds(base, _CB)], didx[b])
        return pltpu.async_copy(g_hbm.at[sidx[b]], rows[b], sems[b])

    pending = load_and_fire(0, 0)
    for j in range(nchunks):
        b = j % 2
        pending.wait()
        if j + 1 < nchunks:
            pending = load_and_fire(j + 1, (j + 1) % 2)
        pltpu.sync_copy(rows[b], acc_sh.at[didx[b]], add=True)

    plsc.subcore_barrier()

    @pl.when(s < 10)
    def _():
        rbase = pl.multiple_of(s * 1000, 8)
        obase = pl.multiple_of(c * _N + s * 1000, 8)
        pltpu.sync_copy(acc_sh.at[pl.ds(rbase, 1000)],
                        out_hbm.at[pl.ds(obase, 1000)])


def _sc_edge_agg(g, src, dst, zeros_n32):
    k = pl.kernel(
        _agg_body,
        out_type=jax.ShapeDtypeStruct((_NC * _N, 32), jnp.float32),
        mesh=_sc_mesh,
        scratch_types=[
            pltpu.VMEM((_CB,), jnp.int32),
            pltpu.VMEM((_CB,), jnp.int32),
            pltpu.VMEM((_CB,), jnp.int32),
            pltpu.VMEM((_CB,), jnp.int32),
            pltpu.VMEM((_CB, 32), jnp.float32),
            pltpu.VMEM((_CB, 32), jnp.float32),
            pltpu.VMEM_SHARED((_N, 32), jnp.float32),
            pltpu.SemaphoreType.DMA,
            pltpu.SemaphoreType.DMA,
        ],
        compiler_params=pltpu.CompilerParams(use_tc_tiling_on_sc=False),
    )
    return k(g, src, dst, zeros_n32)


def _dec_body(ucT_hbm, unT_hbm, src_hbm, dst_hbm, ewc_hbm, ewn_hbm,
              sidx_all, didx_all, uT_v, o0, o1, osem0, osem1):
    c = lax.axis_index("c")
    s = lax.axis_index("s")
    wid = c * _NS + s
    ebase = pl.multiple_of(wid * _EPT, 8)

    pltpu.sync_copy(src_hbm.at[pl.ds(ebase, _EPT)], sidx_all)
    pltpu.sync_copy(dst_hbm.at[pl.ds(ebase, _EPT)], didx_all)

    nchunks = _EPT // _CD
    outs = (o0, o1)
    osem = (osem0, osem1)
    out_pending = [None, None]

    for uT_hbm, ew_hbm in ((ucT_hbm, ewc_hbm), (unT_hbm, ewn_hbm)):
        pltpu.sync_copy(uT_hbm, uT_v)
        for j in range(nchunks):
            b = j % 2
            if out_pending[b] is not None:
                out_pending[b].wait()

            def group(g, carry, j=j, b=b):
                si = sidx_all[pl.ds(j * _CD + g * 16, 16)]
                di = didx_all[pl.ds(j * _CD + g * 16, 16)]
                acc = jnp.zeros((16,), jnp.float32)
                for kcol in range(10):
                    rowk = jnp.full((16,), kcol, jnp.int32)
                    av = plsc.load_gather(uT_v, [rowk, si])
                    bv = plsc.load_gather(uT_v, [rowk, di])
                    acc = acc + av * bv
                outs[b][pl.ds(g * 16, 16)] = 1.0 / (1.0 + jnp.exp(-acc))
                return carry

            lax.fori_loop(0, _CD // 16, group, 0)
            obase = pl.multiple_of(ebase + j * _CD, 8)
            out_pending[b] = pltpu.async_copy(
                outs[b], ew_hbm.at[pl.ds(obase, _CD)], osem[b])
        for b in range(2):
            if out_pending[b] is not None:
                out_pending[b].wait()
                out_pending[b] = None


def _sc_decoder(ucT, unT, src, dst):
    k = pl.kernel(
        _dec_body,
        out_type=(jax.ShapeDtypeStruct((_E,), jnp.float32),
                  jax.ShapeDtypeStruct((_E,), jnp.float32)),
        mesh=_sc_mesh,
        scratch_types=[
            pltpu.VMEM((_EPT,), jnp.int32),
            pltpu.VMEM((_EPT,), jnp.int32),
            pltpu.VMEM((10, _N), jnp.float32),
            pltpu.VMEM((_CD,), jnp.float32),
            pltpu.VMEM((_CD,), jnp.float32),
            pltpu.SemaphoreType.DMA,
            pltpu.SemaphoreType.DMA,
        ],
        compiler_params=pltpu.CompilerParams(use_tc_tiling_on_sc=False,
                                             needs_layout_passes=False),
    )
    return k(ucT, unT, src, dst)


_BLK = 1000
_GRID = _N // _BLK


def _row_spec(d):
    return pl.BlockSpec((_BLK, d), lambda i: (i, 0))


def _full_spec(r, ccol):
    return pl.BlockSpec((r, ccol), lambda i: (0, 0))


def _part_spec(d):
    return pl.BlockSpec((_BLK, d), lambda i: (i, 0))


def _part_spec_hi(d):
    return pl.BlockSpec((_BLK, d), lambda i: (i + _GRID, 0))


def _tc12_body(x_ref, w_ref, d0_ref, d1_ref, h0_ref, dinv_ref, g0_ref):
    deg = d0_ref[:, 0:1] + d1_ref[:, 0:1] + 1.0
    dinv = 1.0 / jnp.sqrt(deg)
    h0 = jnp.dot(x_ref[...], w_ref[...], preferred_element_type=jnp.float32)
    h0_ref[...] = h0
    dinv_ref[...] = dinv
    g0_ref[...] = h0 * dinv


def _tc12(x, w, degp):
    return pl.pallas_call(
        _tc12_body,
        grid=(_GRID,),
        in_specs=[_row_spec(128), _full_spec(128, 32),
                  _part_spec(16), _part_spec_hi(16)],
        out_specs=[_row_spec(32), _row_spec(1), _row_spec(32)],
        out_shape=[jax.ShapeDtypeStruct((_N, 32), jnp.float32),
                   jax.ShapeDtypeStruct((_N, 1), jnp.float32),
                   jax.ShapeDtypeStruct((_N, 32), jnp.float32)],
    )(x, w, degp, degp)


def _tc3_body(p0_ref, p1_ref, h0_ref, dinv_ref, b_ref, h_ref, g1_ref):
    dinv = dinv_ref[...]
    agg = dinv * (p0_ref[...] + p1_ref[...]) + dinv * dinv * h0_ref[...]
    h = jnp.maximum(agg + b_ref[...], 0.0)
    h_ref[...] = h
    g1_ref[...] = dinv * h


def _tc3(aggp, h0, dinv, b):
    return pl.pallas_call(
        _tc3_body,
        grid=(_GRID,),
        in_specs=[_part_spec(32), _part_spec_hi(32), _row_spec(32), _row_spec(1),
                  _full_spec(1, 32)],
        out_specs=[_row_spec(32), _row_spec(32)],
        out_shape=[jax.ShapeDtypeStruct((_N, 32), jnp.float32),
                   jax.ShapeDtypeStruct((_N, 32), jnp.float32)],
    )(aggp, aggp, h0, dinv, b)


def _tc4_body(p0_ref, p1_ref, h_ref, dinv_ref,
              wmc_ref, bmc_ref, wmn_ref, bmn_ref,
              wlc_ref, blc_ref, wln_ref, bln_ref,
              ec_ref, en_ref,
              cw1_ref, cb1_ref, cw2_ref, cb2_ref,
              nw1_ref, nb1_ref, nw2_ref, nb2_ref,
              muc_ref, mun_ref, lvc_ref, lvn_ref, ucp_ref, unp_ref):
    dinv = dinv_ref[...]
    agg = dinv * (p0_ref[...] + p1_ref[...]) + dinv * dinv * h_ref[...]

    def mm(a, w_ref, b_ref):
        return jnp.dot(a, w_ref[...], preferred_element_type=jnp.float32) + b_ref[...]

    muc = mm(agg, wmc_ref, bmc_ref)
    mun = mm(agg, wmn_ref, bmn_ref)
    lvc = mm(agg, wlc_ref, blc_ref)
    lvn = mm(agg, wln_ref, bln_ref)
    muc_ref[...] = muc
    mun_ref[...] = mun
    lvc_ref[...] = lvc
    lvn_ref[...] = lvn
    zc = muc + ec_ref[...] * jnp.exp(0.5 * lvc)
    zn = mun + en_ref[...] * jnp.exp(0.5 * lvn)
    tc = jnp.tanh(mm(zc, cw1_ref, cb1_ref))
    tn = jnp.tanh(mm(zn, nw1_ref, nb1_ref))
    ucp_ref[...] = jnp.tanh(mm(tc, cw2_ref, cb2_ref))
    unp_ref[...] = jnp.tanh(mm(tn, nw2_ref, nb2_ref))


def _tc4(aggp, h, dinv, wmc, bmc, wmn, bmn, wlc, blc, wln, bln,
         ec, en, cw1, cb1, cw2, cb2, nw1, nb1, nw2, nb2):
    tspec = _row_spec(16)
    return pl.pallas_call(
        _tc4_body,
        grid=(_GRID,),
        in_specs=[_part_spec(32), _part_spec_hi(32), _row_spec(32), _row_spec(1),
                  _full_spec(32, 64), _full_spec(1, 64),
                  _full_spec(32, 64), _full_spec(1, 64),
                  _full_spec(32, 64), _full_spec(1, 64),
                  _full_spec(32, 64), _full_spec(1, 64),
                  _row_spec(64), _row_spec(64),
                  _full_spec(64, 32), _full_spec(1, 32),
                  _full_spec(32, 16), _full_spec(1, 16),
                  _full_spec(64, 32), _full_spec(1, 32),
                  _full_spec(32, 16), _full_spec(1, 16)],
        out_specs=[_row_spec(64), _row_spec(64), _row_spec(64), _row_spec(64),
                   tspec, tspec],
        out_shape=[jax.ShapeDtypeStruct((_N, 64), jnp.float32),
                   jax.ShapeDtypeStruct((_N, 64), jnp.float32),
                   jax.ShapeDtypeStruct((_N, 64), jnp.float32),
                   jax.ShapeDtypeStruct((_N, 64), jnp.float32),
                   jax.ShapeDtypeStruct((_N, 16), jnp.float32),
                   jax.ShapeDtypeStruct((_N, 16), jnp.float32)],
    )(aggp, aggp, h, dinv, wmc, bmc, wmn, bmn, wlc, blc, wln, bln,
      ec, en, cw1, cb1, cw2, cb2, nw1, nb1, nw2, nb2)


def kernel(x, edge_index, W_shared, b_shared, W_mu_c, b_mu_c, W_mu_nc, b_mu_nc,
           W_lv_c, b_lv_c, W_lv_nc, b_lv_nc,
           dc_fcW, dc_fcb, dc_fc2W, dc_fc2b,
           dn_fcW, dn_fcb, dn_fc2W, dn_fc2b):
    src, dst = edge_index[0], edge_index[1]

    eps_c = jax.random.normal(jax.random.key(7), (_N, 64), jnp.float32)
    eps_nc = jax.random.normal(jax.random.key(8), (_N, 64), jnp.float32)

    zeros_n16 = jnp.zeros((_N, 16), jnp.float32)
    zeros_n32 = jnp.zeros((_N, 32), jnp.float32)

    degp = _sc_degree(dst, zeros_n16)

    h0, dinv, g0 = _tc12(x, W_shared, degp)

    agg1p = _sc_edge_agg(g0, src, dst, zeros_n32)
    h, g1 = _tc3(agg1p, h0, dinv, b_shared.reshape(1, 32))

    agg2p = _sc_edge_agg(g1, src, dst, zeros_n32)

    pad_w = lambda w: jnp.pad(w, ((0, 0), (0, 6)))
    pad_b = lambda b: jnp.pad(b, ((0, 6),)).reshape(1, 16)

    mu_c, mu_nc, lv_c, lv_nc, ucp, unp = _tc4(
        agg2p, h, dinv,
        W_mu_c, b_mu_c.reshape(1, 64), W_mu_nc, b_mu_nc.reshape(1, 64),
        W_lv_c, b_lv_c.reshape(1, 64), W_lv_nc, b_lv_nc.reshape(1, 64),
        eps_c, eps_nc,
        dc_fcW, dc_fcb.reshape(1, 32), pad_w(dc_fc2W), pad_b(dc_fc2b),
        dn_fcW, dn_fcb.reshape(1, 32), pad_w(dn_fc2W), pad_b(dn_fc2b))

    ew_c, ew_nc = _sc_decoder(jnp.transpose(ucp[:, :10]),
                              jnp.transpose(unp[:, :10]), src, dst)
    return (ew_c, ew_nc, mu_c, mu_nc, lv_c, lv_nc)
